# baseline (speedup 1.0000x reference)
import functools

import jax
import jax.numpy as jnp
import numpy as np
from jax.experimental import pallas as pl
from jax.experimental.pallas import tpu as pltpu
from jax.experimental.pallas import tpu_sc as plsc

B = 2
C_IN = 64
H = 96
W = 320
HEAD_CONV = 256
NUM_CLASS = 3
KDET = 50
C_ROI = C_IN + 2 + NUM_CLASS
HW = H * W


def _nms_body(h_ref, o_ref):
    x = h_ref[...]
    ninf = jnp.float32(-jnp.inf)
    up = jnp.concatenate([x[:, :, 1:, :], jnp.full((B, 3, 1, W), ninf)], axis=2)
    dn = jnp.concatenate([jnp.full((B, 3, 1, W), ninf), x[:, :, :-1, :]], axis=2)
    m1 = jnp.maximum(jnp.maximum(x, up), dn)
    lf = jnp.concatenate([m1[:, :, :, 1:], jnp.full((B, 3, H, 1), ninf)], axis=3)
    rt = jnp.concatenate([jnp.full((B, 3, H, 1), ninf), m1[:, :, :, :-1]], axis=3)
    hmax = jnp.maximum(jnp.maximum(m1, lf), rt)
    o_ref[...] = x * (hmax == x).astype(x.dtype)


def _nms_pallas(h):
    return pl.pallas_call(
        _nms_body,
        out_shape=jax.ShapeDtypeStruct(h.shape, h.dtype),
    )(h)


def _k2_body(hm_ref, o2d_ref, s2d_ref, o_ref, s_ref):
    x = hm_ref[0]
    ninf = jnp.float32(-jnp.inf)
    up = jnp.concatenate([x[:, 1:, :], jnp.full((3, 1, W), ninf)], axis=1)
    dn = jnp.concatenate([jnp.full((3, 1, W), ninf), x[:, :-1, :]], axis=1)
    m1 = jnp.maximum(jnp.maximum(x, up), dn)
    lf = jnp.concatenate([m1[:, :, 1:], jnp.full((3, H, 1), ninf)], axis=2)
    rt = jnp.concatenate([jnp.full((3, H, 1), ninf), m1[:, :, :-1]], axis=2)
    hmax = jnp.maximum(jnp.maximum(m1, lf), rt)
    s_ref[...] = x * (hmax == x).astype(x.dtype)

    ci = jax.lax.broadcasted_iota(jnp.int32, (3, H, W), 0)
    yi = jax.lax.broadcasted_iota(jnp.int32, (3, H, W), 1)
    xi = jax.lax.broadcasted_iota(jnp.int32, (3, H, W), 2)
    flat = ci * HW + yi * W + xi
    yx = yi[0] * W + xi[0]
    lane = jax.lax.broadcasted_iota(jnp.int32, (8, 128), 1)
    row = jax.lax.broadcasted_iota(jnp.int32, (8, 128), 0)
    big = jnp.int32(2**31 - 1)

    def body(i, acc):
        cur = s_ref[...]
        v = jnp.max(cur)
        j = jnp.min(jnp.where(cur == v, flat, big))
        s_ref[...] = jnp.where(flat == j, ninf, cur)
        pos = j % HW
        m2 = (yx == pos)
        off_x = jnp.sum(jnp.where(m2, o2d_ref[0, 0], 0.0))
        off_y = jnp.sum(jnp.where(m2, o2d_ref[0, 1], 0.0))
        sw = jnp.sum(jnp.where(m2, s2d_ref[0, 0], 0.0))
        sh = jnp.sum(jnp.where(m2, s2d_ref[0, 1], 0.0))
        cx = (pos % W).astype(jnp.float32) + off_x
        cy = (pos // W).astype(jnp.float32) + off_y
        vals = (jnp.where(row == 0, cx - sw / 2.0, 0.0)
                + jnp.where(row == 1, cy - sh / 2.0, 0.0)
                + jnp.where(row == 2, cx + sw / 2.0, 0.0)
                + jnp.where(row == 3, cy + sh / 2.0, 0.0)
                + jnp.where(row == 4, (j // HW).astype(jnp.float32), 0.0))
        return jnp.where(lane == i, vals, acc)

    o_ref[0] = jax.lax.fori_loop(0, KDET, body, jnp.zeros((8, 128), jnp.float32))


def _select_pallas(heatmap, offset_2d, size_2d):
    out = pl.pallas_call(
        _k2_body,
        grid=(B,),
        in_specs=[
            pl.BlockSpec((1, 3, H, W), lambda b: (b, 0, 0, 0)),
            pl.BlockSpec((1, 2, H, W), lambda b: (b, 0, 0, 0)),
            pl.BlockSpec((1, 2, H, W), lambda b: (b, 0, 0, 0)),
        ],
        out_specs=pl.BlockSpec((1, 8, 128), lambda b: (b, 0, 0)),
        out_shape=jax.ShapeDtypeStruct((B, 8, 128), jnp.float32),
        scratch_shapes=[pltpu.VMEM((3, H, W), jnp.float32)],
        compiler_params=pltpu.CompilerParams(
            dimension_semantics=("parallel",)),
    )(heatmap, offset_2d, size_2d)
    dets = out[:, :5, :KDET]
    bids = jnp.broadcast_to(jnp.arange(B, dtype=jnp.float32)[:, None], (B, KDET))
    box = jnp.concatenate([bids[:, None, :], dets[:, :4]], axis=1)
    box = box.transpose(0, 2, 1).reshape(B * KDET, 5)
    cls_ids = dets[:, 4].reshape(B * KDET).astype(jnp.int32)
    return box, cls_ids


_K1_ROWS = 8
_WP = 384
_XF_COLS = (H + 2) * _WP + 128


def _k1_body(x_ref, w1_ref, b1_ref, w2_ref, b2_ref, o_ref):
    t = pl.program_id(1)
    wide = _K1_ROWS * _WP
    slabs = []
    for dy in range(3):
        start = pl.multiple_of((t * _K1_ROWS + dy) * _WP, 128)
        slab = x_ref[0, :, pl.ds(start, wide + 128)]
        for dx in range(3):
            slabs.append(slab[:, dx:dx + wide])
    rhs = jnp.concatenate(slabs, axis=0)
    a = jnp.dot(w1_ref[...], rhs, preferred_element_type=jnp.float32)
    a = jnp.maximum(a + b1_ref[...], 0.0)
    z = jnp.dot(w2_ref[...], a, preferred_element_type=jnp.float32) + b2_ref[...]
    for r in range(_K1_ROWS):
        o_ref[0, :, r * W:(r + 1) * W] = z[:, r * _WP:r * _WP + W]


def _dense_heads_pallas(XF, W1T, b1, W2T, b2):
    return pl.pallas_call(
        _k1_body,
        grid=(B, H // _K1_ROWS),
        in_specs=[
            pl.BlockSpec((1, 64, _XF_COLS), lambda b, t: (b, 0, 0)),
            pl.BlockSpec((768, 576), lambda b, t: (0, 0)),
            pl.BlockSpec((768, 1), lambda b, t: (0, 0)),
            pl.BlockSpec((8, 768), lambda b, t: (0, 0)),
            pl.BlockSpec((8, 1), lambda b, t: (0, 0)),
        ],
        out_specs=pl.BlockSpec((1, 8, _K1_ROWS * W), lambda b, t: (b, 0, t)),
        out_shape=jax.ShapeDtypeStruct((B, 8, HW), jnp.float32),
        compiler_params=pltpu.CompilerParams(
            dimension_semantics=("parallel", "parallel")),
    )(XF, W1T, b1, W2T, b2)


def _dense_heads(features, p):
    xp = jnp.pad(features, ((0, 0), (0, 0), (1, 1), (1, 1), ))
    xp = jnp.pad(xp, ((0, 0), (0, 0), (0, 0), (0, _WP - W - 2)))
    XF = jnp.pad(xp.reshape(B, 64, (H + 2) * _WP), ((0, 0), (0, 0), (0, 128)))
    w1 = jnp.concatenate([p['hm_w1'], p['o2d_w1'], p['s2d_w1']], axis=0)
    W1T = w1.transpose(0, 2, 3, 1).reshape(768, 576)
    b1 = jnp.concatenate([p['hm_b1'], p['o2d_b1'], p['s2d_b1']])[:, None]
    W2T = jnp.zeros((8, 768), jnp.float32)
    W2T = W2T.at[0:3, 0:256].set(p['hm_w2'][:, :, 0, 0])
    W2T = W2T.at[3:5, 256:512].set(p['o2d_w2'][:, :, 0, 0])
    W2T = W2T.at[5:7, 512:768].set(p['s2d_w2'][:, :, 0, 0])
    b2 = jnp.concatenate([p['hm_b2'], p['o2d_b2'], p['s2d_b2'],
                          jnp.zeros((1,), jnp.float32)])[:, None]
    dense = _dense_heads_pallas(XF, W1T, b1, W2T, b2)
    heatmap = dense[:, 0:3, :].reshape(B, 3, H, W)
    offset_2d = dense[:, 3:5, :].reshape(B, 2, H, W)
    size_2d = dense[:, 5:7, :].reshape(B, 2, H, W)
    return heatmap, offset_2d, size_2d


_K4_TB = 10
_K4_GRID = (B * KDET) // _K4_TB
_K4_WIDE = _K4_TB * 128
_XR_COLS = B * KDET * 128 + 128


def _k4_body(x_ref, w1_ref, b1_ref, m_ref, w2_ref, b2_ref, o_ref, acc_ref):
    t = pl.program_id(0)
    start = pl.multiple_of(t * _K4_WIDE, 128)
    loaded = x_ref[:, pl.ds(start, _K4_WIDE + 128)]
    slabs = []
    for dy in range(3):
        for dx in range(3):
            off = dy * 9 + dx
            slabs.append(loaded[:, off:off + _K4_WIDE])
    rhs = jnp.concatenate(slabs, axis=0)
    a = jnp.dot(w1_ref[...], rhs, preferred_element_type=jnp.float32)
    a = jnp.maximum(a + b1_ref[...], 0.0)
    part = jnp.dot(a, m_ref[0], preferred_element_type=jnp.float32)

    @pl.when(t == 0)
    def _():
        acc_ref[...] = part

    @pl.when(t > 0)
    def _():
        acc_ref[...] = acc_ref[...] + part

    @pl.when(t == _K4_GRID - 1)
    def _():
        o_ref[...] = (jnp.dot(w2_ref[...], acc_ref[...],
                              preferred_element_type=jnp.float32) + b2_ref[...])


def _roi_heads_pallas(XR, W1f, b1f, M, W2f, b2f):
    return pl.pallas_call(
        _k4_body,
        grid=(_K4_GRID,),
        in_specs=[
            pl.BlockSpec((69, _XR_COLS), lambda t: (0, 0)),
            pl.BlockSpec((1024, 621), lambda t: (0, 0)),
            pl.BlockSpec((1024, 1), lambda t: (0, 0)),
            pl.BlockSpec((1, _K4_WIDE, 128), lambda t: (t, 0, 0)),
            pl.BlockSpec((32, 1024), lambda t: (0, 0)),
            pl.BlockSpec((32, 1), lambda t: (0, 0)),
        ],
        out_specs=pl.BlockSpec((32, 128), lambda t: (0, 0)),
        out_shape=jax.ShapeDtypeStruct((32, 128), jnp.float32),
        scratch_shapes=[pltpu.VMEM((1024, 128), jnp.float32)],
    )(XR, W1f, b1f, M, W2f, b2f)


_K4_HEADS = (('dep', 0, 2), ('o3d', 2, 4), ('s3d', 4, 8), ('hd', 8, 32))


def _k4_mean_mask():
    m = np.zeros((_K4_GRID, _K4_WIDE, 128), np.float32)
    for t in range(_K4_GRID):
        for nloc in range(_K4_TB):
            n = t * _K4_TB + nloc
            for r in range(7):
                for c in range(7):
                    m[t, nloc * 128 + r * 9 + c, n] = 1.0 / 49.0
    return jnp.asarray(m)


def _roi_heads(roi_in, p):
    N = B * KDET
    xp = jnp.pad(roi_in, ((0, 0), (0, 0), (1, 1), (1, 1)))
    xp = jnp.pad(xp.reshape(N, C_ROI, 81), ((0, 0), (0, 0), (0, 47)))
    XR = jnp.pad(xp.transpose(1, 0, 2).reshape(C_ROI, N * 128),
                 ((0, 0), (0, 128)))
    w1s, b1s = [], []
    for name, lo, hi in _K4_HEADS:
        s = p[name + '_bn_g'] / jnp.sqrt(p[name + '_bn_v'] + 1e-5)
        w1s.append((p[name + '_w1'] * s[:, None, None, None])
                   .transpose(0, 2, 3, 1).reshape(HEAD_CONV, 621))
        b1s.append((p[name + '_b1'] - p[name + '_bn_m']) * s + p[name + '_bn_b'])
    W1f = jnp.concatenate(w1s, axis=0)
    b1f = jnp.concatenate(b1s)[:, None]
    W2f = jnp.zeros((32, 1024), jnp.float32)
    b2s = []
    for i, (name, lo, hi) in enumerate(_K4_HEADS):
        W2f = W2f.at[lo:hi, i * HEAD_CONV:(i + 1) * HEAD_CONV].set(
            p[name + '_w2'][:, :, 0, 0])
        b2s.append(p[name + '_b2'])
    b2f = jnp.concatenate(b2s)[:, None]
    z = _roi_heads_pallas(XR, W1f, b1f, _k4_mean_mask(), W2f, b2f)[:, :N]
    return z[0:2].T, z[2:4].T, z[4:8].T, z[8:32].T


_SC_NC = 2
_SC_NS = 16
_SC_NW = _SC_NC * _SC_NS
_NPTS = B * KDET * 49
_PPAD = 4928
_BP = 4 * _PPAD
_BPW = _BP // _SC_NW
_DP = 128


def _k3_gather(table, idx):
    mesh = plsc.VectorSubcoreMesh(core_axis_name="c", subcore_axis_name="s")

    @functools.partial(
        pl.kernel, mesh=mesh,
        out_type=jax.ShapeDtypeStruct((_BP, _DP), jnp.float32),
        scratch_types=[
            pltpu.VMEM((_BPW,), jnp.int32),
            pltpu.VMEM((_BPW, _DP), jnp.float32),
            pltpu.SemaphoreType.DMA,
        ],
    )
    def k(table_hbm, idx_hbm, out_hbm, idx_v, rows_v, sem):
        wid = jax.lax.axis_index("s") * _SC_NC + jax.lax.axis_index("c")
        base = wid * _BPW
        pltpu.sync_copy(idx_hbm.at[pl.ds(base, _BPW)], idx_v)
        pltpu.async_copy(table_hbm.at[idx_v], rows_v, sem).wait()
        pltpu.sync_copy(rows_v, out_hbm.at[pl.ds(base, _BPW)])

    return k(table, idx)


def _k3_blend_body(r_ref, w_ref, o_ref):
    o_ref[...] = jnp.sum(r_ref[...] * w_ref[...], axis=0)


def _k3_blend(rows, wts):
    return pl.pallas_call(
        _k3_blend_body,
        out_shape=jax.ShapeDtypeStruct((_PPAD, _DP), jnp.float32),
    )(rows.reshape(4, _PPAD, _DP), wts)


def _roi_align_sc(features, box):
    table = jnp.pad(features.transpose(0, 2, 3, 1),
                    ((0, 0), (0, 0), (0, 0), (0, _DP - C_IN))).reshape(
                        B * HW, _DP)
    bidx = box[:, 0].astype(jnp.int32)
    g = (jnp.arange(7, dtype=jnp.float32) + 0.5) / 7.0
    xs = box[:, 1:2] + g[None, :] * (box[:, 3:4] - box[:, 1:2]) - 0.5
    ys = box[:, 2:3] + g[None, :] * (box[:, 4:5] - box[:, 2:3]) - 0.5
    x0 = jnp.floor(xs)
    y0 = jnp.floor(ys)
    wx = xs - x0
    wy = ys - y0
    x0i = jnp.clip(x0.astype(jnp.int32), 0, W - 1)
    x1i = jnp.clip(x0.astype(jnp.int32) + 1, 0, W - 1)
    y0i = jnp.clip(y0.astype(jnp.int32), 0, H - 1)
    y1i = jnp.clip(y0.astype(jnp.int32) + 1, 0, H - 1)
    base = (bidx * HW)[:, None, None]
    ry0 = (y0i * W)[:, :, None]
    ry1 = (y1i * W)[:, :, None]
    cx0 = x0i[:, None, :]
    cx1 = x1i[:, None, :]
    i00 = (base + ry0 + cx0).reshape(_NPTS)
    i01 = (base + ry0 + cx1).reshape(_NPTS)
    i10 = (base + ry1 + cx0).reshape(_NPTS)
    i11 = (base + ry1 + cx1).reshape(_NPTS)
    wyk = wy[:, :, None]
    wxk = wx[:, None, :]
    ones7 = jnp.ones((B * KDET, 7, 7), jnp.float32)
    w00 = ((1 - wxk) * (1 - wyk) * ones7).reshape(_NPTS)
    w01 = (wxk * (1 - wyk) * ones7).reshape(_NPTS)
    w10 = ((1 - wxk) * wyk * ones7).reshape(_NPTS)
    w11 = (wxk * wyk * ones7).reshape(_NPTS)
    pad = _PPAD - _NPTS
    idx = jnp.concatenate([jnp.pad(i, (0, pad)) for i in (i00, i01, i10, i11)])
    wts = jnp.stack([jnp.pad(w, (0, pad)) for w in (w00, w01, w10, w11)])
    rows = _k3_gather(table, idx)
    blended = _k3_blend(rows, wts[:, :, None])[:_NPTS, :C_IN]
    return blended.reshape(B * KDET, 49, C_IN).transpose(0, 2, 1).reshape(
        B * KDET, C_IN, 7, 7)


def _conv(x, w, b, pad):
    y = jax.lax.conv_general_dilated(x, w, (1, 1), pad,
                                     dimension_numbers=('NCHW', 'OIHW', 'NCHW'))
    return y + b[None, :, None, None]


def _dense_head(x, p, name):
    h = jax.nn.relu(_conv(x, p[name + '_w1'], p[name + '_b1'], 'SAME'))
    return _conv(h, p[name + '_w2'], p[name + '_b2'], 'VALID')


def _roi_head(x, p, name):
    h = _conv(x, p[name + '_w1'], p[name + '_b1'], 'SAME')
    h = (h - p[name + '_bn_m'][None, :, None, None]) / jnp.sqrt(p[name + '_bn_v'][None, :, None, None] + 1e-5)
    h = h * p[name + '_bn_g'][None, :, None, None] + p[name + '_bn_b'][None, :, None, None]
    h = jax.nn.relu(h)
    h = jnp.mean(h, axis=(2, 3), keepdims=True)
    return _conv(h, p[name + '_w2'], p[name + '_b2'], 'VALID')


def _select_topk(heat, K):
    b, c, hh, ww = heat.shape
    flat = heat.reshape(b, c, hh * ww)
    s_all, i_all = jax.lax.top_k(flat, K)
    scores, inds = jax.lax.top_k(s_all.reshape(b, c * K), K)
    clses = inds // K
    inds_all = jnp.take_along_axis(i_all.reshape(b, c * K), inds, axis=1)
    return scores, inds_all, clses


def _bilinear(img, xs, ys):
    x0 = jnp.floor(xs)
    y0 = jnp.floor(ys)
    wx = xs - x0
    wy = ys - y0
    x0i = jnp.clip(x0.astype(jnp.int32), 0, W - 1)
    x1i = jnp.clip(x0.astype(jnp.int32) + 1, 0, W - 1)
    y0i = jnp.clip(y0.astype(jnp.int32), 0, H - 1)
    y1i = jnp.clip(y0.astype(jnp.int32) + 1, 0, H - 1)
    Ia = img[:, y0i, x0i]
    Ib = img[:, y0i, x1i]
    Ic = img[:, y1i, x0i]
    Id = img[:, y1i, x1i]
    return Ia * (1 - wx) * (1 - wy) + Ib * wx * (1 - wy) + Ic * (1 - wx) * wy + Id * wx * wy


def _roi_align(feat, boxes):
    def one(box):
        bidx = box[0].astype(jnp.int32)
        g = (jnp.arange(7, dtype=jnp.float32) + 0.5) / 7.0
        xs = box[1] + g * (box[3] - box[1])
        ys = box[2] + g * (box[4] - box[2])
        xg, yg = jnp.meshgrid(xs, ys)
        return _bilinear(feat[bidx], xg - 0.5, yg - 0.5)
    return jax.vmap(one)(boxes)


def _project(calib, pts):
    cu = calib[:, 0, 2]
    cv = calib[:, 1, 2]
    fu = calib[:, 0, 0]
    fv = calib[:, 1, 1]
    bx = calib[:, 0, 3] / (-fu)
    by = calib[:, 1, 3] / (-fv)
    x = (pts[:, 0] - cu) * pts[:, 2] / fu + bx
    y = (pts[:, 1] - cv) * pts[:, 2] / fv + by
    return jnp.stack([x, y, pts[:, 2]], -1)


def kernel(features, calib, coord_range, params):
    p = params
    heatmap, offset_2d, size_2d = _dense_heads(features, p)
    box, cls_ids = _select_pallas(heatmap, offset_2d, size_2d)
    roi_feat = _roi_align_sc(features, box)
    bidx = box[:, 0].astype(jnp.int32)
    cr = coord_range[bidx]
    sx = cr[:, 1, 0] - cr[:, 0, 0]
    ox = cr[:, 0, 0]
    sy = cr[:, 1, 1] - cr[:, 0, 1]
    oy = cr[:, 0, 1]
    box_s = jnp.stack([box[:, 0], box[:, 1] / W * sx + ox, box[:, 2] / H * sy + oy,
                       box[:, 3] / W * sx + ox, box[:, 4] / H * sy + oy], -1)
    roi_calib = calib[bidx]
    N = B * KDET
    ones = jnp.ones((N, 1), dtype=jnp.float32)
    p1 = _project(roi_calib, jnp.concatenate([box_s[:, 1:3], ones], -1))[:, :2]
    p2 = _project(roi_calib, jnp.concatenate([box_s[:, 3:5], ones], -1))[:, :2]
    cic = jnp.concatenate([box_s[:, 0:1], p1, p2], -1)
    t = jnp.arange(7, dtype=jnp.float32) / 6.0
    cx = cic[:, 1:2] + t[None, :] * (cic[:, 3:4] - cic[:, 1:2])
    cy = cic[:, 2:3] + t[None, :] * (cic[:, 4:5] - cic[:, 2:3])
    coord_maps = jnp.concatenate([
        jnp.broadcast_to(cx[:, None, None, :], (N, 1, 7, 7)),
        jnp.broadcast_to(cy[:, None, :, None], (N, 1, 7, 7))], 1)
    cls_hot = jax.nn.one_hot(cls_ids, NUM_CLASS, dtype=jnp.float32)
    roi_in = jnp.concatenate([roi_feat, coord_maps,
                              jnp.broadcast_to(cls_hot[:, :, None, None], (N, NUM_CLASS, 7, 7))], 1)
    box2d_h = jnp.clip(box_s[:, 4] - box_s[:, 2], 1.0, None)
    dnet, offset_3d, s3d, heading = _roi_heads(roi_in, p)
    h3d_log_std = s3d[:, 3:4]
    size_3d = p['mean_size'][cls_ids] + s3d[:, :3]
    depth_geo = size_3d[:, 0] / box2d_h * roi_calib[:, 0, 0]
    dgls = (h3d_log_std[:, 0] + 2.0 * (jnp.log(roi_calib[:, 0, 0]) - jnp.log(box2d_h)))[:, None]
    dnls = jax.nn.logsumexp(jnp.concatenate([dnet[:, 1:2], dgls], -1), axis=-1, keepdims=True)
    depth = jnp.concatenate([1.0 / (jax.nn.sigmoid(dnet[:, 0:1]) + 1e-6) - 1.0 + depth_geo[:, None], dnls], -1)
    return heatmap, offset_2d, size_2d, heading, depth, offset_3d, size_3d


# speedup vs baseline: 1.0025x; 1.0025x over previous
import functools

import jax
import jax.numpy as jnp
import numpy as np
from jax.experimental import pallas as pl
from jax.experimental.pallas import tpu as pltpu
from jax.experimental.pallas import tpu_sc as plsc

B = 2
C_IN = 64
H = 96
W = 320
HEAD_CONV = 256
NUM_CLASS = 3
KDET = 50
C_ROI = C_IN + 2 + NUM_CLASS
HW = H * W


def _k2_body(hm_ref, o2d_ref, s2d_ref, o_ref, s_ref):
    x = hm_ref[0]
    ninf = jnp.float32(-jnp.inf)
    up = jnp.concatenate([x[:, 1:, :], jnp.full((3, 1, W), ninf)], axis=1)
    dn = jnp.concatenate([jnp.full((3, 1, W), ninf), x[:, :-1, :]], axis=1)
    m1 = jnp.maximum(jnp.maximum(x, up), dn)
    lf = jnp.concatenate([m1[:, :, 1:], jnp.full((3, H, 1), ninf)], axis=2)
    rt = jnp.concatenate([jnp.full((3, H, 1), ninf), m1[:, :, :-1]], axis=2)
    hmax = jnp.maximum(jnp.maximum(m1, lf), rt)
    s_ref[...] = x * (hmax == x).astype(x.dtype)

    ci = jax.lax.broadcasted_iota(jnp.int32, (3, H, W), 0)
    yi = jax.lax.broadcasted_iota(jnp.int32, (3, H, W), 1)
    xi = jax.lax.broadcasted_iota(jnp.int32, (3, H, W), 2)
    flat = ci * HW + yi * W + xi
    yx = yi[0] * W + xi[0]
    lane = jax.lax.broadcasted_iota(jnp.int32, (8, 128), 1)
    row = jax.lax.broadcasted_iota(jnp.int32, (8, 128), 0)
    big = jnp.int32(2**31 - 1)

    def body(i, acc):
        cur = s_ref[...]
        v = jnp.max(cur)
        j = jnp.min(jnp.where(cur == v, flat, big))
        s_ref[...] = jnp.where(flat == j, ninf, cur)
        pos = j % HW
        m2 = (yx == pos)
        off_x = jnp.sum(jnp.where(m2, o2d_ref[0, 0], 0.0))
        off_y = jnp.sum(jnp.where(m2, o2d_ref[0, 1], 0.0))
        sw = jnp.sum(jnp.where(m2, s2d_ref[0, 0], 0.0))
        sh = jnp.sum(jnp.where(m2, s2d_ref[0, 1], 0.0))
        cx = (pos % W).astype(jnp.float32) + off_x
        cy = (pos // W).astype(jnp.float32) + off_y
        vals = (jnp.where(row == 0, cx - sw / 2.0, 0.0)
                + jnp.where(row == 1, cy - sh / 2.0, 0.0)
                + jnp.where(row == 2, cx + sw / 2.0, 0.0)
                + jnp.where(row == 3, cy + sh / 2.0, 0.0)
                + jnp.where(row == 4, (j // HW).astype(jnp.float32), 0.0))
        return jnp.where(lane == i, vals, acc)

    o_ref[0] = jax.lax.fori_loop(0, KDET, body, jnp.zeros((8, 128), jnp.float32))


def _select_pallas(heatmap, offset_2d, size_2d):
    out = pl.pallas_call(
        _k2_body,
        grid=(B,),
        in_specs=[
            pl.BlockSpec((1, 3, H, W), lambda b: (b, 0, 0, 0)),
            pl.BlockSpec((1, 2, H, W), lambda b: (b, 0, 0, 0)),
            pl.BlockSpec((1, 2, H, W), lambda b: (b, 0, 0, 0)),
        ],
        out_specs=pl.BlockSpec((1, 8, 128), lambda b: (b, 0, 0)),
        out_shape=jax.ShapeDtypeStruct((B, 8, 128), jnp.float32),
        scratch_shapes=[pltpu.VMEM((3, H, W), jnp.float32)],
        compiler_params=pltpu.CompilerParams(
            dimension_semantics=("parallel",)),
    )(heatmap, offset_2d, size_2d)
    dets = out[:, :5, :KDET]
    bids = jnp.broadcast_to(jnp.arange(B, dtype=jnp.float32)[:, None], (B, KDET))
    box = jnp.concatenate([bids[:, None, :], dets[:, :4]], axis=1)
    box = box.transpose(0, 2, 1).reshape(B * KDET, 5)
    cls_ids = dets[:, 4].reshape(B * KDET).astype(jnp.int32)
    return box, cls_ids


_K1_ROWS = 8
_WP = 384
_XF_COLS = (H + 2) * _WP + 128


def _k1_body(x_ref, w1_ref, b1_ref, w2_ref, b2_ref, o_ref):
    t = pl.program_id(1)
    wide = _K1_ROWS * _WP
    slabs = []
    for dy in range(3):
        start = pl.multiple_of((t * _K1_ROWS + dy) * _WP, 128)
        slab = x_ref[0, :, pl.ds(start, wide + 128)]
        for dx in range(3):
            slabs.append(slab[:, dx:dx + wide])
    rhs = jnp.concatenate(slabs, axis=0)
    a = jnp.dot(w1_ref[...], rhs, preferred_element_type=jnp.float32)
    a = jnp.maximum(a + b1_ref[...], 0.0)
    z = jnp.dot(w2_ref[...], a, preferred_element_type=jnp.float32) + b2_ref[...]
    for r in range(_K1_ROWS):
        o_ref[0, :, r * W:(r + 1) * W] = z[:, r * _WP:r * _WP + W]


def _dense_heads_pallas(XF, W1T, b1, W2T, b2):
    return pl.pallas_call(
        _k1_body,
        grid=(B, H // _K1_ROWS),
        in_specs=[
            pl.BlockSpec((1, 64, _XF_COLS), lambda b, t: (b, 0, 0)),
            pl.BlockSpec((768, 576), lambda b, t: (0, 0)),
            pl.BlockSpec((768, 1), lambda b, t: (0, 0)),
            pl.BlockSpec((8, 768), lambda b, t: (0, 0)),
            pl.BlockSpec((8, 1), lambda b, t: (0, 0)),
        ],
        out_specs=pl.BlockSpec((1, 8, _K1_ROWS * W), lambda b, t: (b, 0, t)),
        out_shape=jax.ShapeDtypeStruct((B, 8, HW), jnp.float32),
        compiler_params=pltpu.CompilerParams(
            dimension_semantics=("parallel", "parallel")),
    )(XF, W1T, b1, W2T, b2)


def _dense_heads(features, p):
    xp = jnp.pad(features, ((0, 0), (0, 0), (1, 1), (1, 1), ))
    xp = jnp.pad(xp, ((0, 0), (0, 0), (0, 0), (0, _WP - W - 2)))
    XF = jnp.pad(xp.reshape(B, 64, (H + 2) * _WP), ((0, 0), (0, 0), (0, 128)))
    w1 = jnp.concatenate([p['hm_w1'], p['o2d_w1'], p['s2d_w1']], axis=0)
    W1T = w1.transpose(0, 2, 3, 1).reshape(768, 576)
    b1 = jnp.concatenate([p['hm_b1'], p['o2d_b1'], p['s2d_b1']])[:, None]
    W2T = jnp.zeros((8, 768), jnp.float32)
    W2T = W2T.at[0:3, 0:256].set(p['hm_w2'][:, :, 0, 0])
    W2T = W2T.at[3:5, 256:512].set(p['o2d_w2'][:, :, 0, 0])
    W2T = W2T.at[5:7, 512:768].set(p['s2d_w2'][:, :, 0, 0])
    b2 = jnp.concatenate([p['hm_b2'], p['o2d_b2'], p['s2d_b2'],
                          jnp.zeros((1,), jnp.float32)])[:, None]
    dense = _dense_heads_pallas(XF, W1T, b1, W2T, b2)
    heatmap = dense[:, 0:3, :].reshape(B, 3, H, W)
    offset_2d = dense[:, 3:5, :].reshape(B, 2, H, W)
    size_2d = dense[:, 5:7, :].reshape(B, 2, H, W)
    return heatmap, offset_2d, size_2d


_K4_TB = 10
_K4_GRID = (B * KDET) // _K4_TB
_K4_WIDE = _K4_TB * 128
_XR_COLS = B * KDET * 128 + 128


def _k4_body(x_ref, w1_ref, b1_ref, m_ref, w2_ref, b2_ref, o_ref, acc_ref):
    t = pl.program_id(0)
    start = pl.multiple_of(t * _K4_WIDE, 128)
    loaded = x_ref[:, pl.ds(start, _K4_WIDE + 128)]
    slabs = []
    for dy in range(3):
        for dx in range(3):
            off = dy * 9 + dx
            slabs.append(loaded[:, off:off + _K4_WIDE])
    rhs = jnp.concatenate(slabs, axis=0)
    a = jnp.dot(w1_ref[...], rhs, preferred_element_type=jnp.float32)
    a = jnp.maximum(a + b1_ref[...], 0.0)
    part = jnp.dot(a, m_ref[0], preferred_element_type=jnp.float32)

    @pl.when(t == 0)
    def _():
        acc_ref[...] = part

    @pl.when(t > 0)
    def _():
        acc_ref[...] = acc_ref[...] + part

    @pl.when(t == _K4_GRID - 1)
    def _():
        o_ref[...] = (jnp.dot(w2_ref[...], acc_ref[...],
                              preferred_element_type=jnp.float32) + b2_ref[...])


def _roi_heads_pallas(XR, W1f, b1f, M, W2f, b2f):
    return pl.pallas_call(
        _k4_body,
        grid=(_K4_GRID,),
        in_specs=[
            pl.BlockSpec((69, _XR_COLS), lambda t: (0, 0)),
            pl.BlockSpec((1024, 621), lambda t: (0, 0)),
            pl.BlockSpec((1024, 1), lambda t: (0, 0)),
            pl.BlockSpec((1, _K4_WIDE, 128), lambda t: (t, 0, 0)),
            pl.BlockSpec((32, 1024), lambda t: (0, 0)),
            pl.BlockSpec((32, 1), lambda t: (0, 0)),
        ],
        out_specs=pl.BlockSpec((32, 128), lambda t: (0, 0)),
        out_shape=jax.ShapeDtypeStruct((32, 128), jnp.float32),
        scratch_shapes=[pltpu.VMEM((1024, 128), jnp.float32)],
    )(XR, W1f, b1f, M, W2f, b2f)


_K4_HEADS = (('dep', 0, 2), ('o3d', 2, 4), ('s3d', 4, 8), ('hd', 8, 32))


def _k4_mean_mask():
    m = np.zeros((_K4_GRID, _K4_WIDE, 128), np.float32)
    for t in range(_K4_GRID):
        for nloc in range(_K4_TB):
            n = t * _K4_TB + nloc
            for r in range(7):
                for c in range(7):
                    m[t, nloc * 128 + r * 9 + c, n] = 1.0 / 49.0
    return jnp.asarray(m)


def _roi_heads(roi_in, p):
    N = B * KDET
    xp = jnp.pad(roi_in, ((0, 0), (0, 0), (1, 1), (1, 1)))
    xp = jnp.pad(xp.reshape(N, C_ROI, 81), ((0, 0), (0, 0), (0, 47)))
    XR = jnp.pad(xp.transpose(1, 0, 2).reshape(C_ROI, N * 128),
                 ((0, 0), (0, 128)))
    w1s, b1s = [], []
    for name, lo, hi in _K4_HEADS:
        s = p[name + '_bn_g'] / jnp.sqrt(p[name + '_bn_v'] + 1e-5)
        w1s.append((p[name + '_w1'] * s[:, None, None, None])
                   .transpose(0, 2, 3, 1).reshape(HEAD_CONV, 621))
        b1s.append((p[name + '_b1'] - p[name + '_bn_m']) * s + p[name + '_bn_b'])
    W1f = jnp.concatenate(w1s, axis=0)
    b1f = jnp.concatenate(b1s)[:, None]
    W2f = jnp.zeros((32, 1024), jnp.float32)
    b2s = []
    for i, (name, lo, hi) in enumerate(_K4_HEADS):
        W2f = W2f.at[lo:hi, i * HEAD_CONV:(i + 1) * HEAD_CONV].set(
            p[name + '_w2'][:, :, 0, 0])
        b2s.append(p[name + '_b2'])
    b2f = jnp.concatenate(b2s)[:, None]
    z = _roi_heads_pallas(XR, W1f, b1f, _k4_mean_mask(), W2f, b2f)[:, :N]
    return z[0:2].T, z[2:4].T, z[4:8].T, z[8:32].T


_SC_NC = 2
_SC_NS = 16
_SC_NW = _SC_NC * _SC_NS
_NPTS = B * KDET * 49
_PPAD = 4928
_BP = 4 * _PPAD
_BPW = _BP // _SC_NW
_DP = 128


def _k3_gather(table, idx):
    mesh = plsc.VectorSubcoreMesh(core_axis_name="c", subcore_axis_name="s")

    @functools.partial(
        pl.kernel, mesh=mesh,
        out_type=jax.ShapeDtypeStruct((_BP, _DP), jnp.float32),
        scratch_types=[
            pltpu.VMEM((_BPW,), jnp.int32),
            pltpu.VMEM((_BPW, _DP), jnp.float32),
            pltpu.SemaphoreType.DMA,
        ],
    )
    def k(table_hbm, idx_hbm, out_hbm, idx_v, rows_v, sem):
        wid = jax.lax.axis_index("s") * _SC_NC + jax.lax.axis_index("c")
        base = wid * _BPW
        pltpu.sync_copy(idx_hbm.at[pl.ds(base, _BPW)], idx_v)
        pltpu.async_copy(table_hbm.at[idx_v], rows_v, sem).wait()
        pltpu.sync_copy(rows_v, out_hbm.at[pl.ds(base, _BPW)])

    return k(table, idx)


def _k3_blend_body(r_ref, w_ref, o_ref):
    o_ref[...] = jnp.sum(r_ref[...] * w_ref[...], axis=0)


def _k3_blend(rows, wts):
    return pl.pallas_call(
        _k3_blend_body,
        out_shape=jax.ShapeDtypeStruct((_PPAD, _DP), jnp.float32),
    )(rows.reshape(4, _PPAD, _DP), wts)


def _roi_align_sc(features, box):
    table = jnp.pad(features.transpose(0, 2, 3, 1),
                    ((0, 0), (0, 0), (0, 0), (0, _DP - C_IN))).reshape(
                        B * HW, _DP)
    bidx = box[:, 0].astype(jnp.int32)
    g = (jnp.arange(7, dtype=jnp.float32) + 0.5) / 7.0
    xs = box[:, 1:2] + g[None, :] * (box[:, 3:4] - box[:, 1:2]) - 0.5
    ys = box[:, 2:3] + g[None, :] * (box[:, 4:5] - box[:, 2:3]) - 0.5
    x0 = jnp.floor(xs)
    y0 = jnp.floor(ys)
    wx = xs - x0
    wy = ys - y0
    x0i = jnp.clip(x0.astype(jnp.int32), 0, W - 1)
    x1i = jnp.clip(x0.astype(jnp.int32) + 1, 0, W - 1)
    y0i = jnp.clip(y0.astype(jnp.int32), 0, H - 1)
    y1i = jnp.clip(y0.astype(jnp.int32) + 1, 0, H - 1)
    base = (bidx * HW)[:, None, None]
    ry0 = (y0i * W)[:, :, None]
    ry1 = (y1i * W)[:, :, None]
    cx0 = x0i[:, None, :]
    cx1 = x1i[:, None, :]
    i00 = (base + ry0 + cx0).reshape(_NPTS)
    i01 = (base + ry0 + cx1).reshape(_NPTS)
    i10 = (base + ry1 + cx0).reshape(_NPTS)
    i11 = (base + ry1 + cx1).reshape(_NPTS)
    wyk = wy[:, :, None]
    wxk = wx[:, None, :]
    ones7 = jnp.ones((B * KDET, 7, 7), jnp.float32)
    w00 = ((1 - wxk) * (1 - wyk) * ones7).reshape(_NPTS)
    w01 = (wxk * (1 - wyk) * ones7).reshape(_NPTS)
    w10 = ((1 - wxk) * wyk * ones7).reshape(_NPTS)
    w11 = (wxk * wyk * ones7).reshape(_NPTS)
    pad = _PPAD - _NPTS
    idx = jnp.concatenate([jnp.pad(i, (0, pad)) for i in (i00, i01, i10, i11)])
    wts = jnp.stack([jnp.pad(w, (0, pad)) for w in (w00, w01, w10, w11)])
    rows = _k3_gather(table, idx)
    blended = _k3_blend(rows, wts[:, :, None])[:_NPTS, :C_IN]
    return blended.reshape(B * KDET, 49, C_IN).transpose(0, 2, 1).reshape(
        B * KDET, C_IN, 7, 7)


def _project(calib, pts):
    cu = calib[:, 0, 2]
    cv = calib[:, 1, 2]
    fu = calib[:, 0, 0]
    fv = calib[:, 1, 1]
    bx = calib[:, 0, 3] / (-fu)
    by = calib[:, 1, 3] / (-fv)
    x = (pts[:, 0] - cu) * pts[:, 2] / fu + bx
    y = (pts[:, 1] - cv) * pts[:, 2] / fv + by
    return jnp.stack([x, y, pts[:, 2]], -1)


def kernel(features, calib, coord_range, params):
    p = params
    heatmap, offset_2d, size_2d = _dense_heads(features, p)
    box, cls_ids = _select_pallas(heatmap, offset_2d, size_2d)
    roi_feat = _roi_align_sc(features, box)
    bidx = box[:, 0].astype(jnp.int32)
    cr = coord_range[bidx]
    sx = cr[:, 1, 0] - cr[:, 0, 0]
    ox = cr[:, 0, 0]
    sy = cr[:, 1, 1] - cr[:, 0, 1]
    oy = cr[:, 0, 1]
    box_s = jnp.stack([box[:, 0], box[:, 1] / W * sx + ox, box[:, 2] / H * sy + oy,
                       box[:, 3] / W * sx + ox, box[:, 4] / H * sy + oy], -1)
    roi_calib = calib[bidx]
    N = B * KDET
    ones = jnp.ones((N, 1), dtype=jnp.float32)
    p1 = _project(roi_calib, jnp.concatenate([box_s[:, 1:3], ones], -1))[:, :2]
    p2 = _project(roi_calib, jnp.concatenate([box_s[:, 3:5], ones], -1))[:, :2]
    cic = jnp.concatenate([box_s[:, 0:1], p1, p2], -1)
    t = jnp.arange(7, dtype=jnp.float32) / 6.0
    cx = cic[:, 1:2] + t[None, :] * (cic[:, 3:4] - cic[:, 1:2])
    cy = cic[:, 2:3] + t[None, :] * (cic[:, 4:5] - cic[:, 2:3])
    coord_maps = jnp.concatenate([
        jnp.broadcast_to(cx[:, None, None, :], (N, 1, 7, 7)),
        jnp.broadcast_to(cy[:, None, :, None], (N, 1, 7, 7))], 1)
    cls_hot = jax.nn.one_hot(cls_ids, NUM_CLASS, dtype=jnp.float32)
    roi_in = jnp.concatenate([roi_feat, coord_maps,
                              jnp.broadcast_to(cls_hot[:, :, None, None], (N, NUM_CLASS, 7, 7))], 1)
    box2d_h = jnp.clip(box_s[:, 4] - box_s[:, 2], 1.0, None)
    dnet, offset_3d, s3d, heading = _roi_heads(roi_in, p)
    h3d_log_std = s3d[:, 3:4]
    size_3d = p['mean_size'][cls_ids] + s3d[:, :3]
    depth_geo = size_3d[:, 0] / box2d_h * roi_calib[:, 0, 0]
    dgls = (h3d_log_std[:, 0] + 2.0 * (jnp.log(roi_calib[:, 0, 0]) - jnp.log(box2d_h)))[:, None]
    dnls = jax.nn.logsumexp(jnp.concatenate([dnet[:, 1:2], dgls], -1), axis=-1, keepdims=True)
    depth = jnp.concatenate([1.0 / (jax.nn.sigmoid(dnet[:, 0:1]) + 1e-6) - 1.0 + depth_geo[:, None], dnls], -1)
    return heatmap, offset_2d, size_2d, heading, depth, offset_3d, size_3d
